# baseline (speedup 1.0000x reference)
import jax
import jax.numpy as jnp
from jax import lax
from jax.experimental import pallas as pl
from jax.experimental.pallas import tpu as pltpu
from jax.experimental.pallas import tpu_sc as plsc

N = 10000
D = 256
DH = 128
NC = 2
NS = 16
CH = 128
L = 16

E = 160000
CHUNKS = -(-E // (NS * CH))
E_PAD = CHUNKS * NS * CH
N_PAD = 10240
ROWS_PER_TILE = N_PAD // NS


_RB = 400
_NB = N // _RB


def _mm_split_body(x_ref, w1_ref, w3_ref, h_ref, s_ref):
    xb = x_ref[...]
    h_ref[0] = jnp.dot(xb, w1_ref[...], preferred_element_type=jnp.float32)
    s_ref[0] = jnp.dot(xb, w3_ref[...], preferred_element_type=jnp.float32)


def _tc_layer0(x, W1, W3):
    return pl.pallas_call(
        _mm_split_body,
        grid=(_NB, NC),
        in_specs=[
            pl.BlockSpec((_RB, D), lambda r, c: (r, 0)),
            pl.BlockSpec((D, DH), lambda r, c: (0, c)),
            pl.BlockSpec((D, DH), lambda r, c: (0, c)),
        ],
        out_specs=[
            pl.BlockSpec((1, _RB, DH), lambda r, c: (c, r, 0)),
            pl.BlockSpec((1, _RB, DH), lambda r, c: (c, r, 0)),
        ],
        out_shape=[
            jax.ShapeDtypeStruct((NC, N_PAD, DH), jnp.float32),
            jax.ShapeDtypeStruct((NC, N_PAD, DH), jnp.float32),
        ],
    )(x, W1, W3)


def _relu_mm_body(a_ref, w2_ref, h_ref):
    l1 = jnp.concatenate([jax.nn.relu(a_ref[0]), jax.nn.relu(a_ref[1])], axis=-1)
    h_ref[0] = jnp.dot(l1, w2_ref[...], preferred_element_type=jnp.float32)


def _tc_layer1(a1, W2):
    return pl.pallas_call(
        _relu_mm_body,
        grid=(_NB, NC),
        in_specs=[
            pl.BlockSpec((NC, _RB, DH), lambda r, c: (0, r, 0)),
            pl.BlockSpec((D, DH), lambda r, c: (0, c)),
        ],
        out_specs=pl.BlockSpec((1, _RB, DH), lambda r, c: (c, r, 0)),
        out_shape=jax.ShapeDtypeStruct((NC, N_PAD, DH), jnp.float32),
    )(a1, W2)


def _skip_relu_body(a_ref, s_ref, o_ref):
    o_ref[...] = jax.nn.relu(a_ref[0] + s_ref[0])


def _tc_final(a2, s):
    return pl.pallas_call(
        _skip_relu_body,
        grid=(_NB, NC),
        in_specs=[
            pl.BlockSpec((1, _RB, DH), lambda r, c: (c, r, 0)),
            pl.BlockSpec((1, _RB, DH), lambda r, c: (c, r, 0)),
        ],
        out_specs=pl.BlockSpec((_RB, DH), lambda r, c: (r, c)),
        out_shape=jax.ShapeDtypeStruct((N, D), jnp.float32),
    )(a2, s)


def _spmm_body(tab_ref, src_ref, dst_ref, w_ref, out_ref,
               acc, idx_s, idx_d, w_v, rows, sem):
    c = lax.axis_index("c")
    s = lax.axis_index("s")

    def _zero_row(r, _):
        for j in range(DH // L):
            rows[r, pl.ds(j * L, L)] = jnp.zeros((L,), jnp.float32)
        return 0
    lax.fori_loop(0, CH, _zero_row, 0)
    base = s * ROWS_PER_TILE
    for k in range(ROWS_PER_TILE // CH):
        pltpu.sync_copy(rows, acc.at[pl.ds(base + k * CH, CH)])

    pltpu.sync_copy(src_ref.at[s], idx_s)
    pltpu.sync_copy(dst_ref.at[s], idx_d)
    pltpu.sync_copy(w_ref.at[s], w_v)

    plsc.subcore_barrier()

    def _chunk(g, _):
        pltpu.async_copy(tab_ref.at[c].at[idx_s.at[g]], rows, sem).wait()

        def _group(i, _):
            wv = w_v[g, pl.ds(i * L, L)]
            for l in range(L):
                wb = lax.gather(
                    wv, jnp.full((L, 1), l, jnp.int32),
                    dimension_numbers=lax.GatherDimensionNumbers(
                        offset_dims=(), collapsed_slice_dims=(0,),
                        start_index_map=(0,)),
                    slice_sizes=(1,),
                    mode=lax.GatherScatterMode.PROMISE_IN_BOUNDS)
                e = i * L + l
                for j in range(DH // L):
                    sl = pl.ds(j * L, L)
                    rows[e, sl] = rows[e, sl] * wb
            return 0
        lax.fori_loop(0, CH // L, _group, 0)

        pltpu.sync_copy(rows, acc.at[idx_d.at[g]], add=True)
        return 0

    lax.fori_loop(0, CHUNKS, _chunk, 0)

    plsc.subcore_barrier()

    pltpu.sync_copy(acc.at[pl.ds(base, ROWS_PER_TILE)],
                    out_ref.at[c].at[pl.ds(base, ROWS_PER_TILE)])


_spmm_sc = pl.kernel(
    _spmm_body,
    out_type=jax.ShapeDtypeStruct((NC, N_PAD, DH), jnp.float32),
    mesh=plsc.VectorSubcoreMesh(core_axis_name="c", subcore_axis_name="s",
                                num_cores=NC, num_subcores=NS),
    scratch_types=[
        pltpu.VMEM_SHARED((N_PAD, DH), jnp.float32),
        pltpu.VMEM((CHUNKS, CH), jnp.int32),
        pltpu.VMEM((CHUNKS, CH), jnp.int32),
        pltpu.VMEM((CHUNKS, CH), jnp.float32),
        pltpu.VMEM((CH, DH), jnp.float32),
        pltpu.SemaphoreType.DMA,
    ],
)


def kernel(x, edge_index, edge_weight, W1, W2, W3):
    src = edge_index[0].astype(jnp.int32)
    dst = edge_index[1].astype(jnp.int32)
    w = edge_weight.astype(jnp.float32)
    perm = jnp.argsort((dst >= 5056).astype(jnp.int32), stable=True)
    src = src[perm]
    dst = dst[perm]
    w = w[perm]

    pad = E_PAD - E
    src = jnp.pad(src, (0, pad)).reshape(NS, CHUNKS, CH)
    dst = jnp.pad(dst, (0, pad)).reshape(NS, CHUNKS, CH)
    w = jnp.pad(w, (0, pad)).reshape(NS, CHUNKS, CH)

    h1, s = _tc_layer0(x, W1, W3)
    a1 = _spmm_sc(h1, src, dst, w)
    h2 = _tc_layer1(a1, W2)
    a2 = _spmm_sc(h2, src, dst, w)
    return _tc_final(a2, s)


# speedup vs baseline: 1.3885x; 1.3885x over previous
import jax
import jax.numpy as jnp
from jax import lax
from jax.experimental import pallas as pl
from jax.experimental.pallas import tpu as pltpu
from jax.experimental.pallas import tpu_sc as plsc

N = 10000
D = 256
DH = 128
NC = 2
NS = 16
CH = 128
L = 16

E = 160000
CHUNKS = -(-E // (NS * CH))
E_PAD = CHUNKS * NS * CH
N_PAD = 10240
ROWS_PER_TILE = N_PAD // NS


_RB = 400
_NB = N // _RB


def _mm_split_body(x_ref, w1_ref, w3_ref, h_ref, s_ref):
    xb = x_ref[...]
    h_ref[0] = jnp.dot(xb, w1_ref[...], preferred_element_type=jnp.float32)
    s_ref[0] = jnp.dot(xb, w3_ref[...], preferred_element_type=jnp.float32)


def _tc_layer0(x, W1, W3):
    return pl.pallas_call(
        _mm_split_body,
        grid=(_NB, NC),
        in_specs=[
            pl.BlockSpec((_RB, D), lambda r, c: (r, 0)),
            pl.BlockSpec((D, DH), lambda r, c: (0, c)),
            pl.BlockSpec((D, DH), lambda r, c: (0, c)),
        ],
        out_specs=[
            pl.BlockSpec((1, _RB, DH), lambda r, c: (c, r, 0)),
            pl.BlockSpec((1, _RB, DH), lambda r, c: (c, r, 0)),
        ],
        out_shape=[
            jax.ShapeDtypeStruct((NC, N_PAD, DH), jnp.float32),
            jax.ShapeDtypeStruct((NC, N_PAD, DH), jnp.float32),
        ],
    )(x, W1, W3)


def _relu_mm_body(a_ref, w2_ref, h_ref):
    l1 = jnp.concatenate([jax.nn.relu(a_ref[0]), jax.nn.relu(a_ref[1])], axis=-1)
    h_ref[0] = jnp.dot(l1, w2_ref[...], preferred_element_type=jnp.float32)


def _tc_layer1(a1, W2):
    return pl.pallas_call(
        _relu_mm_body,
        grid=(_NB, NC),
        in_specs=[
            pl.BlockSpec((NC, _RB, DH), lambda r, c: (0, r, 0)),
            pl.BlockSpec((D, DH), lambda r, c: (0, c)),
        ],
        out_specs=pl.BlockSpec((1, _RB, DH), lambda r, c: (c, r, 0)),
        out_shape=jax.ShapeDtypeStruct((NC, N_PAD, DH), jnp.float32),
    )(a1, W2)


def _skip_relu_body(a_ref, s_ref, o_ref):
    o_ref[...] = jax.nn.relu(a_ref[0] + s_ref[0])


def _tc_final(a2, s):
    return pl.pallas_call(
        _skip_relu_body,
        grid=(_NB, NC),
        in_specs=[
            pl.BlockSpec((1, _RB, DH), lambda r, c: (c, r, 0)),
            pl.BlockSpec((1, _RB, DH), lambda r, c: (c, r, 0)),
        ],
        out_specs=pl.BlockSpec((_RB, DH), lambda r, c: (r, c)),
        out_shape=jax.ShapeDtypeStruct((N, D), jnp.float32),
    )(a2, s)


def _spmm_body(tab_ref, src_ref, dst_ref, w_ref, out_ref,
               acc, idx_s, idx_d, w_v, rows, sem):
    c = lax.axis_index("c")
    s = lax.axis_index("s")

    def _zero_row(r, _):
        for j in range(DH // L):
            rows[r, pl.ds(j * L, L)] = jnp.zeros((L,), jnp.float32)
        return 0
    lax.fori_loop(0, CH, _zero_row, 0)
    base = s * ROWS_PER_TILE
    for k in range(ROWS_PER_TILE // CH):
        pltpu.sync_copy(rows, acc.at[pl.ds(base + k * CH, CH)])

    pltpu.sync_copy(src_ref.at[s], idx_s)
    pltpu.sync_copy(dst_ref.at[s], idx_d)
    pltpu.sync_copy(w_ref.at[s], w_v)

    plsc.subcore_barrier()

    def _chunk(g, _):
        pltpu.async_copy(tab_ref.at[c].at[idx_s.at[g]], rows, sem).wait()

        def _group(i, _):
            wv = w_v[g, pl.ds(i * L, L)]
            for l in range(L):
                wb = lax.gather(
                    wv, jnp.full((L, 1), l, jnp.int32),
                    dimension_numbers=lax.GatherDimensionNumbers(
                        offset_dims=(), collapsed_slice_dims=(0,),
                        start_index_map=(0,)),
                    slice_sizes=(1,),
                    mode=lax.GatherScatterMode.PROMISE_IN_BOUNDS)
                e = i * L + l
                for j in range(DH // L):
                    sl = pl.ds(j * L, L)
                    rows[e, sl] = rows[e, sl] * wb
            return 0
        lax.fori_loop(0, CH // L, _group, 0)

        pltpu.sync_copy(rows, acc.at[idx_d.at[g]], add=True)
        return 0

    lax.fori_loop(0, CHUNKS, _chunk, 0)

    plsc.subcore_barrier()

    pltpu.sync_copy(acc.at[pl.ds(base, ROWS_PER_TILE)],
                    out_ref.at[c].at[pl.ds(base, ROWS_PER_TILE)])


_spmm_sc = pl.kernel(
    _spmm_body,
    out_type=jax.ShapeDtypeStruct((NC, N_PAD, DH), jnp.float32),
    mesh=plsc.VectorSubcoreMesh(core_axis_name="c", subcore_axis_name="s",
                                num_cores=NC, num_subcores=NS),
    scratch_types=[
        pltpu.VMEM_SHARED((N_PAD, DH), jnp.float32),
        pltpu.VMEM((CHUNKS, CH), jnp.int32),
        pltpu.VMEM((CHUNKS, CH), jnp.int32),
        pltpu.VMEM((CHUNKS, CH), jnp.float32),
        pltpu.VMEM((CH, DH), jnp.float32),
        pltpu.SemaphoreType.DMA,
    ],
)


def kernel(x, edge_index, edge_weight, W1, W2, W3):
    src = edge_index[0].astype(jnp.int32)
    dst = edge_index[1].astype(jnp.int32)
    w = edge_weight.astype(jnp.float32)

    pad = E_PAD - E
    src = jnp.pad(src, (0, pad)).reshape(NS, CHUNKS, CH)
    dst = jnp.pad(dst, (0, pad)).reshape(NS, CHUNKS, CH)
    w = jnp.pad(w, (0, pad)).reshape(NS, CHUNKS, CH)

    h1, s = _tc_layer0(x, W1, W3)
    a1 = _spmm_sc(h1, src, dst, w)
    h2 = _tc_layer1(a1, W2)
    a2 = _spmm_sc(h2, src, dst, w)
    return _tc_final(a2, s)
